# baseline (speedup 1.0000x reference)
import functools

import jax
import jax.numpy as jnp
from jax import lax
from jax.experimental import pallas as pl
from jax.experimental.pallas import tpu as pltpu
from jax.experimental.pallas import tpu_sc as plsc

B = 64
V = 1000000
NC = 2
NS = 16
L = 16
NW = NC * NS
ROWS_PER_W = B // NW
CH = 20000
NCHUNK = V // CH
NPAIR = NCHUNK // 2
VREGS = CH // L

_mesh = plsc.VectorSubcoreMesh(core_axis_name="c", subcore_axis_name="s")


@functools.partial(
    pl.kernel,
    out_type=jax.ShapeDtypeStruct((B * L,), jnp.int32),
    mesh=_mesh,
    scratch_types=[
        pltpu.VMEM((CH,), jnp.float32),
        pltpu.VMEM((CH,), jnp.float32),
        pltpu.VMEM((L,), jnp.int32),
        pltpu.SemaphoreType.DMA,
        pltpu.SemaphoreType.DMA,
    ],
)
def _argmax_kernel(logits, out, buf0, buf1, outv, sem0, sem1):
    wid = lax.axis_index("c") * NS + lax.axis_index("s")
    lane = lax.iota(jnp.int32, L)

    def scan_chunk(buf, base_vreg, bv, bj):
        def body(j, carry):
            cv, cj = carry
            v = buf[pl.ds(j * L, L)]
            gt = v > cv
            jv = lax.broadcast(base_vreg + j, (L,))
            return jnp.where(gt, v, cv), jnp.where(gt, jv, cj)

        return lax.fori_loop(0, VREGS, body, (bv, bj), unroll=8)

    for r in range(ROWS_PER_W):
        row = wid * ROWS_PER_W + r

        def src(c):
            off = pl.multiple_of(row * V + c * CH, 16)
            return logits.at[pl.ds(off, CH)]

        pltpu.make_async_copy(src(0), buf0, sem0).start()

        def pair_body(cc, carry):
            bv, bj = carry
            c0 = 2 * cc
            pltpu.make_async_copy(src(c0 + 1), buf1, sem1).start()
            pltpu.make_async_copy(src(c0), buf0, sem0).wait()
            bv, bj = scan_chunk(buf0, c0 * VREGS, bv, bj)

            @pl.when(cc < NPAIR - 1)
            def _():
                pltpu.make_async_copy(src(c0 + 2), buf0, sem0).start()

            pltpu.make_async_copy(src(c0 + 1), buf1, sem1).wait()
            bv, bj = scan_chunk(buf1, (c0 + 1) * VREGS, bv, bj)
            return bv, bj

        init = (jnp.full((L,), -jnp.inf, jnp.float32),
                jnp.zeros((L,), jnp.int32))
        bv, bj = lax.fori_loop(0, NPAIR, pair_body, init)

        def perm_gather(x, p):
            return lax.gather(
                x, p[:, None],
                lax.GatherDimensionNumbers(
                    offset_dims=(), collapsed_slice_dims=(0,),
                    start_index_map=(0,)),
                (1,), mode=lax.GatherScatterMode.PROMISE_IN_BOUNDS)

        m = bv
        for s in (1, 2, 4, 8):
            m = jnp.maximum(m, perm_gather(m, lane ^ s))
        idx = bj * L + lane
        cand = jnp.where(bv == m, idx, jnp.int32(2147483647))
        for s in (1, 2, 4, 8):
            cand = jnp.minimum(cand, perm_gather(cand, lane ^ s))
        outv[...] = cand
        pltpu.sync_copy(outv, out.at[pl.ds(pl.multiple_of(row * L, 8), L)])


def kernel(m_logits):
    out = _argmax_kernel(m_logits.reshape(B * V))
    return out.reshape(B, L)[:, :1]


# speedup vs baseline: 49.4787x; 49.4787x over previous
import functools

import jax
import jax.numpy as jnp
from jax import lax
from jax.experimental import pallas as pl
from jax.experimental.pallas import tpu as pltpu
from jax.experimental.pallas import tpu_sc as plsc

B = 64
C = 1000000
L = 16
NS = 16
T0 = 4712
SCB = T0 * 128
TPW = 775
CT = 31
NCH = TPW // CT
CW = CT * 128
VPC = CW // L
TAIL0 = 7812 * 128
TCB = 15872
TNB = SCB // TCB
NEG_INF = float("-inf")
IMAX = 2147483647

_mesh = plsc.VectorSubcoreMesh(core_axis_name="c", subcore_axis_name="s")


@functools.partial(
    pl.kernel,
    out_type=(
        jax.ShapeDtypeStruct((B * L,), jnp.int32),
        jax.ShapeDtypeStruct((B * L,), jnp.float32),
        jax.ShapeDtypeStruct((B * 4 * L,), jnp.float32),
        jax.ShapeDtypeStruct((B * 4 * L,), jnp.int32),
    ),
    mesh=_mesh,
    scratch_types=[
        pltpu.VMEM((8, CW), jnp.float32),
        pltpu.VMEM((8, CW), jnp.float32),
        pltpu.VMEM((8, 64), jnp.float32),
        pltpu.VMEM((8 * L,), jnp.float32),
        pltpu.VMEM((8 * L,), jnp.int32),
        pltpu.VMEM((4 * 8 * L,), jnp.float32),
        pltpu.VMEM((4 * 8 * L,), jnp.int32),
        pltpu.VMEM((L,), jnp.int32),
        pltpu.VMEM((L,), jnp.float32),
        pltpu.SemaphoreType.DMA,
        pltpu.SemaphoreType.DMA,
    ],
)
def _argmax_kernel(logits, out, outval, pv, pi, buf0, buf1, tailb, mv, mi,
                   gv, gi, outrow, outrowv, sem0, sem1):
    cid = lax.axis_index("c")
    sid = lax.axis_index("s")
    wid = cid * NS + sid
    g = wid // 4
    q = wid % 4
    r0 = pl.multiple_of(g * 8, 8)
    cb = pl.multiple_of(SCB + q * TPW * 128, 128)
    lane = lax.iota(jnp.int32, L)

    def src(ch):
        return logits.at[pl.ds(r0, 8), pl.ds(cb + ch * CW, CW)]

    def scan_chunk(buf, step_base, carry):
        def body(jj, c):
            bvs, bjs = list(c[:8]), list(c[8:])
            base = step_base + jj * 8
            jvs = [lax.broadcast(base + c16, (L,)) for c16 in range(8)]
            col0 = jj * 128
            for r in range(8):
                bv, bj = bvs[r], bjs[r]
                for c16 in range(8):
                    v = buf[r, pl.ds(col0 + c16 * L, L)]
                    gt = v > bv
                    bv = jnp.where(gt, v, bv)
                    bj = jnp.where(gt, jvs[c16], bj)
                bvs[r], bjs[r] = bv, bj
            return tuple(bvs) + tuple(bjs)

        return lax.fori_loop(0, CT, body, carry)

    init = tuple(jnp.full((L,), NEG_INF, jnp.float32) for _ in range(8)) + \
        tuple(jnp.zeros((L,), jnp.int32) for _ in range(8))

    pltpu.make_async_copy(src(0), buf0, sem0).start()
    pltpu.make_async_copy(
        logits.at[pl.ds(r0, 8), pl.ds(TAIL0, 64)], tailb, sem1).start()
    pltpu.make_async_copy(
        logits.at[pl.ds(r0, 8), pl.ds(TAIL0, 64)], tailb, sem1).wait()

    def pair_body(cc, carry):
        c0 = 2 * cc
        pltpu.make_async_copy(src(c0 + 1), buf1, sem1).start()
        pltpu.make_async_copy(src(c0), buf0, sem0).wait()
        carry = scan_chunk(buf0, c0 * VPC, carry)
        pltpu.make_async_copy(src(c0 + 2), buf0, sem0).start()
        pltpu.make_async_copy(src(c0 + 1), buf1, sem1).wait()
        carry = scan_chunk(buf1, (c0 + 1) * VPC, carry)
        return carry

    carry = lax.fori_loop(0, (NCH - 1) // 2, pair_body, init)
    pltpu.make_async_copy(src(NCH - 1), buf0, sem0).wait()
    carry = scan_chunk(buf0, (NCH - 1) * VPC, carry)

    bvs, bjs = list(carry[:8]), list(carry[8:])
    qmask = lax.broadcast(
        jnp.where(q == 3, jnp.float32(0.0), jnp.float32(NEG_INF)), (L,))
    tail_base = NCH * VPC
    for r in range(8):
        bv, bj = bvs[r], bjs[r]
        for c16 in range(4):
            v = tailb[r, pl.ds(c16 * L, L)] + qmask
            jv = lax.broadcast(tail_base + c16, (L,))
            gt = v > bv
            bv = jnp.where(gt, v, bv)
            bj = jnp.where(gt, jv, bj)
        bvs[r], bjs[r] = bv, bj

    def perm_gather(x, p):
        return lax.gather(
            x, p[:, None],
            lax.GatherDimensionNumbers(
                offset_dims=(), collapsed_slice_dims=(0,),
                start_index_map=(0,)),
            (1,), mode=lax.GatherScatterMode.PROMISE_IN_BOUNDS)

    for r in range(8):
        bv, bj = bvs[r], bjs[r]
        m = bv
        for s in (1, 2, 4, 8):
            m = jnp.maximum(m, perm_gather(m, lane ^ s))
        idx = cb + bj * L + lane
        cand = jnp.where(bv == m, idx, jnp.int32(IMAX))
        for s in (1, 2, 4, 8):
            cand = jnp.minimum(cand, perm_gather(cand, lane ^ s))
        mv[pl.ds(r * L, L)] = m
        mi[pl.ds(r * L, L)] = cand

    pslot = pl.multiple_of((g * 4 + q) * 8 * L, 16)
    pltpu.sync_copy(mv, pv.at[pl.ds(pslot, 8 * L)])
    pltpu.sync_copy(mi, pi.at[pl.ds(pslot, 8 * L)])
    plsc.subcore_barrier()

    @pl.when(q == 0)
    def _():
        gbase = pl.multiple_of(g * 4 * 8 * L, 16)
        pltpu.sync_copy(pv.at[pl.ds(gbase, 4 * 8 * L)], gv)
        pltpu.sync_copy(pi.at[pl.ds(gbase, 4 * 8 * L)], gi)
        for r in range(8):
            bv = gv[pl.ds(r * L, L)]
            bi = gi[pl.ds(r * L, L)]
            for qq in range(1, 4):
                o = (qq * 8 + r) * L
                v = gv[pl.ds(o, L)]
                i = gi[pl.ds(o, L)]
                take = (v > bv) | ((v == bv) & (i < bi))
                bv = jnp.where(take, v, bv)
                bi = jnp.where(take, i, bi)
            outrow[...] = bi
            outrowv[...] = bv
            row = g * 8 + r
            pltpu.sync_copy(
                outrow, out.at[pl.ds(pl.multiple_of(row * L, 16), L)])
            pltpu.sync_copy(
                outrowv, outval.at[pl.ds(pl.multiple_of(row * L, 16), L)])


def _tc_partial_body(x_ref, val_ref, idx_ref):
    i = pl.program_id(0)
    v = x_ref[...]
    m = jnp.max(v, axis=1, keepdims=True)
    io = lax.broadcasted_iota(jnp.int32, (B, TCB), 1) + i * TCB
    cand = jnp.where(v == m, io, jnp.int32(IMAX))
    bi = jnp.min(cand, axis=1, keepdims=True)

    @pl.when(i == 0)
    def _():
        val_ref[...] = m
        idx_ref[...] = bi

    @pl.when(i > 0)
    def _():
        take = m > val_ref[...]
        val_ref[...] = jnp.where(take, m, val_ref[...])
        idx_ref[...] = jnp.where(take, bi, idx_ref[...])


_tc_partial = pl.pallas_call(
    _tc_partial_body,
    grid=(TNB,),
    in_specs=[pl.BlockSpec((B, TCB), lambda i: (0, i))],
    out_specs=(pl.BlockSpec((B, 1), lambda i: (0, 0)),
               pl.BlockSpec((B, 1), lambda i: (0, 0))),
    out_shape=(jax.ShapeDtypeStruct((B, 1), jnp.float32),
               jax.ShapeDtypeStruct((B, 1), jnp.int32)),
)


def _merge_body(scv_ref, sci_ref, tcv_ref, tci_ref, out_ref):
    scv, sci = scv_ref[...], sci_ref[...]
    tcv, tci = tcv_ref[...], tci_ref[...]
    take = (tcv > scv) | ((tcv == scv) & (tci < sci))
    out_ref[...] = jnp.where(take, tci, sci)


_merge = pl.pallas_call(
    _merge_body,
    out_shape=jax.ShapeDtypeStruct((B, 1), jnp.int32),
)


def kernel(m_logits):
    sc_idx, sc_val, _, _ = _argmax_kernel(m_logits)
    tc_val, tc_idx = _tc_partial(m_logits)
    sci = sc_idx.reshape(B, L)[:, :1]
    scv = sc_val.reshape(B, L)[:, :1]
    return _merge(scv, sci, tc_val, tc_idx)
